# baseline (speedup 1.0000x reference)
import jax
import jax.numpy as jnp
from jax import lax
from jax.experimental import pallas as pl
from jax.experimental.pallas import tpu as pltpu
from jax.experimental.pallas import tpu_sc as plsc

_DIM = 64
_NC, _NS = 2, 16
_NW = _NC * _NS
_G = 128
_GPC = 5
_CHUNK = _G * _GPC


def _make_gather(n_rows):
    per_w = n_rows // _NW
    n_groups = per_w // _G
    n_chunks = n_groups // _GPC
    mesh = plsc.VectorSubcoreMesh(core_axis_name="c", subcore_axis_name="s")

    def body(idx_hbm, table_hbm, out_hbm, idx_v, rows_v, gsem):
        wid = lax.axis_index("s") * _NC + lax.axis_index("c")
        pltpu.sync_copy(idx_hbm.at[wid], idx_v)
        out_base = wid * per_w

        @pl.loop(0, n_chunks)
        def _chunk(c):
            copies = [
                pltpu.async_copy(
                    table_hbm.at[idx_v.at[c * _GPC + g]],
                    rows_v.at[pl.ds(g * _G, _G)],
                    gsem,
                )
                for g in range(_GPC)
            ]
            for cp in copies:
                cp.wait()
            pltpu.sync_copy(
                rows_v, out_hbm.at[pl.ds(out_base + c * _CHUNK, _CHUNK)]
            )

    return pl.kernel(
        body,
        out_type=jax.ShapeDtypeStruct((n_rows, _DIM), jnp.float32),
        mesh=mesh,
        compiler_params=pltpu.CompilerParams(use_tc_tiling_on_sc=False),
        scratch_types=[
            pltpu.VMEM((n_groups, _G), jnp.int32),
            pltpu.VMEM((_CHUNK, _DIM), jnp.float32),
            pltpu.SemaphoreType.DMA,
        ],
    )


def kernel(input_ids, embedding_weight):
    b, l = input_ids.shape
    vocab, dim = embedding_weight.shape
    n = b * l
    assert dim == _DIM and n % (_NW * _G * _GPC) == 0
    idx = input_ids.reshape(_NW, n // (_NW * _G), _G)
    out = _make_gather(n)(idx, embedding_weight)
    return out.reshape(b, l, dim)


# speedup vs baseline: 1.0047x; 1.0047x over previous
import jax
import jax.numpy as jnp
from jax import lax
from jax.experimental import pallas as pl
from jax.experimental.pallas import tpu as pltpu
from jax.experimental.pallas import tpu_sc as plsc

_DIM = 64
_NC, _NS = 2, 16
_NW = _NC * _NS
_G = 128
_GPC = 5
_CHUNK = _G * _GPC


def _make_gather(n_rows):
    per_w = n_rows // _NW
    n_groups = per_w // _G
    n_chunks = n_groups // _GPC
    mesh = plsc.VectorSubcoreMesh(core_axis_name="c", subcore_axis_name="s")

    def body(idx_hbm, table_hbm, out_hbm, idx_v, rows_v, gsem, wsem):
        wid = lax.axis_index("s") * _NC + lax.axis_index("c")
        pltpu.sync_copy(idx_hbm.at[wid], idx_v)
        out_base = wid * per_w

        def fire(c, b):
            for g in range(_GPC):
                pltpu.async_copy(
                    table_hbm.at[idx_v.at[c * _GPC + g]],
                    rows_v.at[b, pl.ds(g * _G, _G)],
                    gsem,
                )

        def drain(sem, b):
            pltpu.make_async_copy(
                out_hbm.at[pl.ds(0, _CHUNK)], rows_v.at[b], sem
            ).wait()

        fire(0, 0)
        @pl.loop(0, n_chunks, step=2)
        def _chunk(c):
            for b in range(2):
                cc = c + b
                ob = 1 - b

                @pl.when(cc > 0)
                def _():
                    drain(wsem, ob)

                @pl.when(cc + 1 < n_chunks)
                def _():
                    fire(cc + 1, ob)

                drain(gsem, b)
                pltpu.async_copy(
                    rows_v.at[b],
                    out_hbm.at[pl.ds(out_base + cc * _CHUNK, _CHUNK)],
                    wsem,
                )

        drain(wsem, 0)

    return pl.kernel(
        body,
        out_type=jax.ShapeDtypeStruct((n_rows, _DIM), jnp.float32),
        mesh=mesh,
        compiler_params=pltpu.CompilerParams(use_tc_tiling_on_sc=False),
        scratch_types=[
            pltpu.VMEM((n_groups, _G), jnp.int32),
            pltpu.VMEM((2, _CHUNK, _DIM), jnp.float32),
            pltpu.SemaphoreType.DMA,
            pltpu.SemaphoreType.DMA,
        ],
    )


def kernel(input_ids, embedding_weight):
    b, l = input_ids.shape
    vocab, dim = embedding_weight.shape
    n = b * l
    assert dim == _DIM and n % (_NW * _G * _GPC) == 0
    idx = input_ids.reshape(_NW, n // (_NW * _G), _G)
    out = _make_gather(n)(idx, embedding_weight)
    return out.reshape(b, l, dim)
